# baseline (speedup 1.0000x reference)
import jax
import jax.numpy as jnp
from jax.experimental import pallas as pl
from jax.experimental.pallas import tpu as pltpu

_B = 1024
_NF = 100000
_D = 64
_H1 = 64
_H2 = 32
_KT = 2048
_NT = _NF // _KT
_REM = _NF - _NT * _KT
_NG = _NT + 1
_NBUF = 4
_EPS = 1e-5

_TDOT = (((0,), (1,)), ((), ()))


def _acc_kernel(x_hbm, e_ref, lw_ref, a_ref, q_ref, xbuf, xtail, sem):
    k = pl.program_id(0)

    def _start(t):
        @pl.when(t < _NT)
        def _full_copy():
            pltpu.make_async_copy(
                x_hbm.at[:, pl.ds(t * _KT, _KT)],
                xbuf.at[jax.lax.rem(t, _NBUF)],
                sem.at[jax.lax.rem(t, _NBUF)]).start()

    @pl.when(k == 0)
    def _prologue():
        a_ref[...] = jnp.zeros_like(a_ref)
        q_ref[...] = jnp.zeros_like(q_ref)
        for t in range(min(_NBUF, _NT)):
            _start(jnp.int32(t))
        pltpu.make_async_copy(
            x_hbm.at[:, pl.ds(_NT * _KT, _REM)], xtail, sem.at[_NBUF]).start()

    slot = jax.lax.rem(k, _NBUF)

    @pl.when(k < _NT)
    def _wait_full():
        pltpu.make_async_copy(
            x_hbm.at[:, pl.ds(k * _KT, _KT)],
            xbuf.at[slot], sem.at[slot]).wait()

    @pl.when(k == _NT)
    def _wait_tail():
        pltpu.make_async_copy(
            x_hbm.at[:, pl.ds(_NT * _KT, _REM)], xtail, sem.at[_NBUF]).wait()

    def _accumulate(x, e, lw):
        aug = jnp.concatenate([e, lw], axis=1)
        a_ref[...] += jax.lax.dot_general(
            aug, x, _TDOT, preferred_element_type=jnp.float32)
        q_ref[...] += jax.lax.dot_general(
            e * e, x * x, _TDOT, preferred_element_type=jnp.float32)

    @pl.when(k < _NT)
    def _full_tile():
        _accumulate(xbuf[slot].astype(jnp.bfloat16),
                    e_ref[...].astype(jnp.bfloat16),
                    lw_ref[...].astype(jnp.bfloat16))

    @pl.when(k == _NT)
    def _tail_tile():
        _accumulate(xtail[...].astype(jnp.bfloat16),
                    e_ref[pl.ds(0, _REM), :].astype(jnp.bfloat16),
                    lw_ref[pl.ds(0, _REM), :].astype(jnp.bfloat16))

    _start(k + _NBUF)


def _bn_t(v, g, b):
    mu = jnp.mean(v, axis=1, keepdims=True)
    var = jnp.mean(jnp.square(v - mu), axis=1, keepdims=True)
    return (v - mu) / jnp.sqrt(var + _EPS) * g + b


def _tail_kernel(a_ref, q_ref, lb_ref, g0_ref, b0_ref,
                 w1_ref, b1_ref, g1_ref, bb1_ref,
                 w2_ref, b2_ref, g2_ref, bb2_ref, hw_ref, out_ref):
    se = a_ref[:_D, :]
    lin = a_ref[_D:_D + 1, :]
    bi = 0.5 * (se * se - q_ref[...])
    z = _bn_t(bi, g0_ref[...], b0_ref[...])
    z = jnp.dot(w1_ref[...], z,
                preferred_element_type=jnp.float32) + b1_ref[...]
    z = jax.nn.relu(_bn_t(z, g1_ref[...], bb1_ref[...]))
    z = jnp.dot(w2_ref[...], z,
                preferred_element_type=jnp.float32) + b2_ref[...]
    z = jax.nn.relu(_bn_t(z, g2_ref[...], bb2_ref[...]))
    y = jnp.sum(z * hw_ref[...], axis=0, keepdims=True)
    out_ref[...] = y + lin + lb_ref[...]


def kernel(feature_values, feature_embed, lin_w, lin_b, bn0_g, bn0_b,
           W1, b1, bn1_g, bn1_b, W2, b2, bn2_g, bn2_b, h_w):
    acc_a, acc_q = pl.pallas_call(
        _acc_kernel,
        grid=(_NG,),
        in_specs=[
            pl.BlockSpec(memory_space=pltpu.MemorySpace.HBM),
            pl.BlockSpec((_KT, _D), lambda k: (k, 0)),
            pl.BlockSpec((_KT, 1), lambda k: (k, 0)),
        ],
        out_specs=[
            pl.BlockSpec((_D + 1, _B), lambda k: (0, 0)),
            pl.BlockSpec((_D, _B), lambda k: (0, 0)),
        ],
        out_shape=[
            jax.ShapeDtypeStruct((_D + 1, _B), jnp.float32),
            jax.ShapeDtypeStruct((_D, _B), jnp.float32),
        ],
        scratch_shapes=[
            pltpu.VMEM((_NBUF, _B, _KT), jnp.float32),
            pltpu.VMEM((_B, _REM), jnp.float32),
            pltpu.SemaphoreType.DMA((_NBUF + 1,)),
        ],
        compiler_params=pltpu.CompilerParams(
            dimension_semantics=("arbitrary",),
        ),
    )(feature_values, feature_embed, lin_w.reshape(_NF, 1))

    out = pl.pallas_call(
        _tail_kernel,
        out_shape=jax.ShapeDtypeStruct((1, _B), jnp.float32),
    )(acc_a, acc_q,
      lin_b.reshape(1, 1), bn0_g.reshape(_D, 1), bn0_b.reshape(_D, 1),
      W1, b1.reshape(_H1, 1), bn1_g.reshape(_H1, 1), bn1_b.reshape(_H1, 1),
      W2, b2.reshape(_H2, 1), bn2_g.reshape(_H2, 1), bn2_b.reshape(_H2, 1),
      h_w.reshape(_H2, 1))
    return out.reshape(_B)
